# Initial kernel scaffold; baseline (speedup 1.0000x reference)
import jax
import jax.numpy as jnp
from jax.experimental import pallas as pl


def kernel(x, W_f, b_f, W_s, b_s, W_o, b_o):
    raise NotImplementedError("write your pallas kernel here")


# speedup vs baseline: 10.0063x; 10.0063x over previous
import functools

import jax
import jax.numpy as jnp
from jax import lax
from jax.experimental import pallas as pl
from jax.experimental.pallas import tpu as pltpu

B, V, F = 4, 2048, 64
K = 40
NDIM, NPROP, NFILT = 4, 64, 128

RBLK = 512
NBLK = V // RBLK


def _grav_kernel(x_ref, Wf_ref, bf_ref, Ws_ref, bs_ref, Wo_ref, bo_ref,
                 out_ref, feat_ref, sel_ref, nmax_ref, nsum_ref):
    rb = pl.program_id(1)
    x_full = x_ref[0]
    feat = jnp.dot(x_full, Wf_ref[...],
                   preferred_element_type=jnp.float32) + bf_ref[...][None, :]
    feat_ref[...] = feat
    coords = jnp.dot(x_full, Ws_ref[...],
                     preferred_element_type=jnp.float32) + bs_ref[...][None, :]
    cn = jnp.sum(coords * coords, axis=1)

    x_blk = x_ref[0, pl.ds(rb * RBLK, RBLK), :]
    c_blk = jnp.dot(x_blk, Ws_ref[...],
                    preferred_element_type=jnp.float32) + bs_ref[...][None, :]
    rn = jnp.sum(c_blk * c_blk, axis=1, keepdims=True)
    cross = lax.dot_general(c_blk, coords, (((1,), (1,)), ((), ())),
                            preferred_element_type=jnp.float32)
    D = rn - 2.0 * cross + cn[None, :]
    sel_ref[...] = D
    nmax_ref[...] = jnp.full((RBLK, NPROP), -jnp.inf, jnp.float32)
    nsum_ref[...] = jnp.zeros((RBLK, NPROP), jnp.float32)

    def body(k, carry):
        s = sel_ref[...]
        m = jnp.min(s, axis=1, keepdims=True)
        H = (s == m).astype(jnp.float32)
        sel_ref[...] = jnp.where(s == m, jnp.inf, s)
        G = jnp.dot(H, feat_ref[...],
                    preferred_element_type=jnp.float32)
        w = jnp.exp(-10.0 * jnp.abs(m))
        wG = w * G
        keep = k > 0
        nmax_ref[...] = jnp.maximum(nmax_ref[...],
                                    jnp.where(keep, wG, -jnp.inf))
        nsum_ref[...] = nsum_ref[...] + jnp.where(keep, wG, 0.0)
        return carry

    lax.fori_loop(0, K, body, 0)

    nmean = nsum_ref[...] * (1.0 / (K - 1))
    acc = (jnp.dot(x_blk, Wo_ref[0:F, :], preferred_element_type=jnp.float32)
           + jnp.dot(nmax_ref[...], Wo_ref[F:F + NPROP, :],
                     preferred_element_type=jnp.float32)
           + jnp.dot(nmean, Wo_ref[F + NPROP:, :],
                     preferred_element_type=jnp.float32)
           + bo_ref[...][None, :])
    out_ref[0] = jnp.tanh(acc)


@jax.jit
def kernel(x, W_f, b_f, W_s, b_s, W_o, b_o):
    grid = (B, NBLK)
    return pl.pallas_call(
        _grav_kernel,
        grid=grid,
        in_specs=[
            pl.BlockSpec((1, V, F), lambda b, r: (b, 0, 0)),
            pl.BlockSpec((F, NPROP), lambda b, r: (0, 0)),
            pl.BlockSpec((NPROP,), lambda b, r: (0,)),
            pl.BlockSpec((F, NDIM), lambda b, r: (0, 0)),
            pl.BlockSpec((NDIM,), lambda b, r: (0,)),
            pl.BlockSpec((F + 2 * NPROP, NFILT), lambda b, r: (0, 0)),
            pl.BlockSpec((NFILT,), lambda b, r: (0,)),
        ],
        out_specs=pl.BlockSpec((1, RBLK, NFILT), lambda b, r: (b, r, 0)),
        out_shape=jax.ShapeDtypeStruct((B, V, NFILT), jnp.float32),
        scratch_shapes=[
            pltpu.VMEM((V, NPROP), jnp.float32),
            pltpu.VMEM((RBLK, V), jnp.float32),
            pltpu.VMEM((RBLK, NPROP), jnp.float32),
            pltpu.VMEM((RBLK, NPROP), jnp.float32),
        ],
    )(x, W_f, b_f, W_s, b_s, W_o, b_o)
